# Initial kernel scaffold; baseline (speedup 1.0000x reference)
import functools

import jax
import jax.numpy as jnp
from jax import lax
from jax.experimental import pallas as pl
from jax.experimental.pallas import tpu as pltpu
from jax.experimental.pallas import tpu_sc as plsc

N = 10000
E = 320000
D = 128
H = 128
NB = E // 128
NSUB = 16
ROWS_A = 640
ROWS_LAST = N - 15 * ROWS_A
F32 = jnp.float32
I32 = jnp.int32

_mesh = plsc.VectorSubcoreMesh(core_axis_name="c", subcore_axis_name="s")


def _rsqrt16(x):
    b = plsc.bitcast(x, I32)
    y = plsc.bitcast(jnp.int32(0x5F3759DF) - (b >> 1), F32)
    for _ in range(3):
        y = y * (1.5 - 0.5 * x * y * y)
    return jnp.where(x > 0, y, 0.0)


def _k_deg(dst2d, ew2d, em2d, zeros1d, dis_out,
           dstbuf, ewbuf, embuf, wbuf, dbuf, disbuf, dacc):
    c = lax.axis_index("c")
    s = lax.axis_index("s")
    cf = c.astype(F32)

    @pl.when(s < 15)
    def _():
        pltpu.sync_copy(zeros1d, dacc.at[pl.ds(s * ROWS_A, ROWS_A)])

    @pl.when(s == 15)
    def _():
        pltpu.sync_copy(zeros1d.at[pl.ds(0, ROWS_LAST)],
                        dacc.at[pl.ds(15 * ROWS_A, ROWS_LAST)])

    plsc.subcore_barrier()

    def body(i, _):
        blk = i * NSUB + s

        @pl.when(blk < NB)
        def _():
            pltpu.sync_copy(dst2d.at[blk], dstbuf)
            pltpu.sync_copy(ew2d.at[blk], ewbuf)
            pltpu.sync_copy(em2d.at[blk], embuf)
            for jj in range(8):
                sl = pl.ds(jj * 16, 16)
                e16 = ewbuf[sl]
                m16 = embuf[sl]
                wbuf[sl] = e16 * ((1.0 - cf) + cf * m16)
            pltpu.sync_copy(wbuf, dacc.at[dstbuf], add=True)
        return 0

    lax.fori_loop(0, (NB + NSUB - 1) // NSUB, body, 0)
    plsc.subcore_barrier()

    def epilogue(base, nrows):
        pltpu.sync_copy(dacc.at[pl.ds(base, nrows)], dbuf.at[pl.ds(0, nrows)])

        def rbody(j, _):
            sl = pl.ds(j * 16, 16)
            disbuf[sl] = _rsqrt16(dbuf[sl])
            return 0

        lax.fori_loop(0, nrows // 16, rbody, 0)
        pltpu.sync_copy(disbuf.at[pl.ds(0, nrows)],
                        dis_out.at[c, pl.ds(base, nrows)])

    @pl.when(s < 15)
    def _():
        epilogue(s * ROWS_A, ROWS_A)

    @pl.when(s == 15)
    def _():
        epilogue(15 * ROWS_A, ROWS_LAST)


@functools.partial(
    pl.kernel,
    out_type=jax.ShapeDtypeStruct((2, N), F32),
    mesh=_mesh,
    scratch_types=[
        pltpu.VMEM((128,), I32),
        pltpu.VMEM((128,), F32),
        pltpu.VMEM((128,), F32),
        pltpu.VMEM((128,), F32),
        pltpu.VMEM((ROWS_A,), F32),
        pltpu.VMEM((ROWS_A,), F32),
        pltpu.VMEM_SHARED((N,), F32),
    ],
)
def k_deg(*refs):
    _k_deg(*refs)


def _k_mat_body(x_ref, wp_ref, out_ref):
    out_ref[...] = jnp.dot(x_ref[...], wp_ref[0],
                           preferred_element_type=F32)


def k_mat(x, wp):
    return pl.pallas_call(
        _k_mat_body,
        grid=(2, 25),
        in_specs=[
            pl.BlockSpec((400, D), lambda c, r: (r, 0)),
            pl.BlockSpec((1, D, 128), lambda c, r: (c, 0, 0)),
        ],
        out_specs=pl.BlockSpec((400, 128), lambda c, r: (c * 25 + r, 0)),
        out_shape=jax.ShapeDtypeStruct((2 * N, 128), F32),
    )(x, wp)


def _k_msg(hcat, src2d, dst2d, ew2d, em2d, dis_all, zeros2d, acc_out,
           srcbuf, dstbuf, soffbuf, ewbuf, embuf, n1buf, n2buf,
           dis_v, dis2_v, rows_v, msg_v, sem, accsp):
    c = lax.axis_index("c")
    s = lax.axis_index("s")

    @pl.when(s < 15)
    def _():
        pltpu.sync_copy(zeros2d, accsp.at[pl.ds(s * ROWS_A, ROWS_A)])

    @pl.when(s == 15)
    def _():
        pltpu.sync_copy(zeros2d.at[pl.ds(0, ROWS_LAST)],
                        accsp.at[pl.ds(15 * ROWS_A, ROWS_LAST)])

    pltpu.sync_copy(dis_all.at[0], dis_v)
    pltpu.sync_copy(dis_all.at[1], dis2_v)
    plsc.subcore_barrier()

    coff = c * N

    def body(i, _):
        blk = i * NSUB + s

        @pl.when(blk < NB)
        def _():
            pltpu.sync_copy(src2d.at[blk], srcbuf)
            pltpu.sync_copy(dst2d.at[blk], dstbuf)
            pltpu.sync_copy(ew2d.at[blk], ewbuf)
            pltpu.sync_copy(em2d.at[blk], embuf)
            for jj in range(8):
                sl = pl.ds(jj * 16, 16)
                soffbuf[sl] = srcbuf[sl] + coff
            pltpu.async_copy(hcat.at[soffbuf], rows_v, sem).wait()
            for jj in range(8):
                sl = pl.ds(jj * 16, 16)
                si = srcbuf[sl]
                di = dstbuf[sl]
                a1 = plsc.load_gather(dis_v, [si])
                b1 = plsc.load_gather(dis_v, [di])
                a2 = plsc.load_gather(dis2_v, [si])
                b2 = plsc.load_gather(dis2_v, [di])
                e16 = ewbuf[sl]
                m16 = embuf[sl]
                n1buf[sl] = a1 * b1 * e16
                n2buf[sl] = a2 * b2 * e16 * m16

            def mbody(k, _):
                kk = jnp.full((16,), k, I32)
                n1 = plsc.load_gather(n1buf, [kk])
                n2 = plsc.load_gather(n2buf, [kk])
                for j in range(4):
                    slj = pl.ds(j * 16, 16)
                    slj1 = pl.ds(64 + j * 16, 16)
                    slj2 = pl.ds(128 + j * 16, 16)
                    rh = rows_v[k, slj]
                    rh1 = rows_v[k, slj1]
                    msg_v[k, slj] = rh * n1
                    msg_v[k, slj1] = rh1 * n1
                    msg_v[k, slj2] = rh * n2
                return 0

            lax.fori_loop(0, 128, mbody, 0)
            pltpu.sync_copy(msg_v, accsp.at[dstbuf], add=True)
        return 0

    lax.fori_loop(0, (NB + NSUB - 1) // NSUB, body, 0)
    plsc.subcore_barrier()

    @pl.when(s < 15)
    def _():
        b = s * ROWS_A
        pltpu.sync_copy(accsp.at[pl.ds(b, ROWS_A)],
                        acc_out.at[c, pl.ds(b, ROWS_A)])

    @pl.when(s == 15)
    def _():
        b = 15 * ROWS_A
        pltpu.sync_copy(accsp.at[pl.ds(b, ROWS_LAST)],
                        acc_out.at[c, pl.ds(b, ROWS_LAST)])


@functools.partial(
    pl.kernel,
    out_type=jax.ShapeDtypeStruct((2, N, 192), F32),
    mesh=_mesh,
    scratch_types=[
        pltpu.VMEM((128,), I32),
        pltpu.VMEM((128,), I32),
        pltpu.VMEM((128,), I32),
        pltpu.VMEM((128,), F32),
        pltpu.VMEM((128,), F32),
        pltpu.VMEM((128,), F32),
        pltpu.VMEM((128,), F32),
        pltpu.VMEM((N,), F32),
        pltpu.VMEM((N,), F32),
        pltpu.VMEM((128, 128), F32),
        pltpu.VMEM((128, 192), F32),
        pltpu.SemaphoreType.DMA,
        pltpu.VMEM_SHARED((N, 192), F32),
    ],
)
def k_msg(*refs):
    _k_msg(*refs)


def _k_out_body(acc_ref, n1_ref, n2_ref, z_ref, z1_ref, z2_ref):
    az = acc_ref[0, :, 0:64]
    az1 = acc_ref[0, :, 64:128]
    az2 = acc_ref[0, :, 128:192]
    z_ref[...] = jnp.maximum(az, 0.0)
    z1_ref[...] = 0.9 * jnp.maximum(az1, 0.0) + 0.1 * n1_ref[...]
    z2_ref[...] = 0.9 * jnp.maximum(az2, 0.0) + 0.1 * n2_ref[...]


def k_out(acc, noise1, noise2):
    o = jax.ShapeDtypeStruct((N, H), F32)
    return pl.pallas_call(
        _k_out_body,
        grid=(2, 25),
        in_specs=[
            pl.BlockSpec((1, 400, 192), lambda c, r: (c, r, 0)),
            pl.BlockSpec((400, 64), lambda c, r: (r, c)),
            pl.BlockSpec((400, 64), lambda c, r: (r, c)),
        ],
        out_specs=[
            pl.BlockSpec((400, 64), lambda c, r: (r, c)),
            pl.BlockSpec((400, 64), lambda c, r: (r, c)),
            pl.BlockSpec((400, 64), lambda c, r: (r, c)),
        ],
        out_shape=(o, o, o),
    )(acc, noise1, noise2)


def kernel(x, edge_index, edge_weight, W):
    kf = jax.random.key(42)
    ka, kb, kn1, kn2 = jax.random.split(kf, 4)
    feat_mask = (jax.random.uniform(ka, (1, D)) > 0.2).astype(x.dtype)
    edge_mask = (jax.random.uniform(kb, (E,)) > 0.2).astype(edge_weight.dtype)
    noise1 = jax.random.normal(kn1, (N, H), dtype=x.dtype)
    noise2 = jax.random.normal(kn2, (N, H), dtype=x.dtype)

    W1 = W * feat_mask[0][:, None]
    wp = jnp.stack([
        jnp.concatenate([W[:, 0:64], W1[:, 0:64]], axis=1),
        jnp.concatenate([W[:, 64:128], W1[:, 64:128]], axis=1),
    ])

    src2d = edge_index[0].astype(I32).reshape(NB, 128)
    dst2d = edge_index[1].astype(I32).reshape(NB, 128)
    ew2d = edge_weight.reshape(NB, 128)
    em2d = edge_mask.reshape(NB, 128)
    zeros1d = jnp.zeros((ROWS_A,), F32)
    zeros2d = jnp.zeros((ROWS_A, 192), F32)

    dis_all = k_deg(dst2d, ew2d, em2d, zeros1d)
    hcat = k_mat(x, wp)
    acc = k_msg(hcat, src2d, dst2d, ew2d, em2d, dis_all, zeros2d)
    z, z1, z2 = k_out(acc, noise1, noise2)
    return (z, z1, z2)


# speedup vs baseline: 5.8905x; 5.8905x over previous
import functools

import jax
import jax.numpy as jnp
from jax import lax
from jax.experimental import pallas as pl
from jax.experimental.pallas import tpu as pltpu
from jax.experimental.pallas import tpu_sc as plsc

N = 10000
E = 320000
D = 128
H = 128
NB = E // 128
NSUB = 16
NP = 10240
ROWS_A = NP // NSUB
RA = 632
F32 = jnp.float32
I32 = jnp.int32

_mesh = plsc.VectorSubcoreMesh(core_axis_name="c", subcore_axis_name="s",
                               num_cores=2, num_subcores=NSUB)


def _rsqrt16(x):
    b = lax.bitcast_convert_type(x, I32)
    y = lax.bitcast_convert_type(jnp.int32(0x5F3759DF) - (b >> 1), F32)
    for _ in range(3):
        y = y * (1.5 - 0.5 * x * y * y)
    return jnp.where(x > 0, y, 0.0)


def _rowshare(s, fn):
    @pl.when(s < 15)
    def _():
        for off, sz in ((0, 128), (128, 128), (256, 128), (384, 128),
                        (512, 120)):
            fn(s * RA + off, sz)

    @pl.when(s == 15)
    def _():
        for off, sz in ((0, 128), (128, 128), (256, 128), (384, 128),
                        (512, 8)):
            fn(15 * RA + off, sz)


def _k_deg(dst2d, ew2d, em2d, zeros1d, dis_out,
           dstbuf, ewbuf, embuf, wbuf, dbuf, disbuf, dacc):
    c = lax.axis_index("c")
    s = lax.axis_index("s")
    cf = c.astype(F32)

    pltpu.sync_copy(zeros1d, dbuf)
    pltpu.sync_copy(dbuf, dacc.at[pl.ds(s * ROWS_A, ROWS_A)])
    plsc.subcore_barrier()

    def body(i, _):
        blk = i * NSUB + s

        @pl.when(blk < NB)
        def _():
            pltpu.sync_copy(dst2d.at[blk], dstbuf)
            pltpu.sync_copy(ew2d.at[blk], ewbuf)
            pltpu.sync_copy(em2d.at[blk], embuf)
            for jj in range(8):
                sl = pl.ds(jj * 16, 16)
                e16 = ewbuf[sl]
                m16 = embuf[sl]
                wbuf[sl] = e16 * ((1.0 - cf) + cf * m16)
            pltpu.sync_copy(wbuf, dacc.at[dstbuf], add=True)
        return 0

    lax.fori_loop(0, (NB + NSUB - 1) // NSUB, body, 0)
    plsc.subcore_barrier()

    base = s * ROWS_A
    pltpu.sync_copy(dacc.at[pl.ds(base, ROWS_A)], dbuf)

    def rbody(j, _):
        sl = pl.ds(j * 16, 16)
        disbuf[sl] = _rsqrt16(dbuf[sl])
        return 0

    lax.fori_loop(0, ROWS_A // 16, rbody, 0)
    pltpu.sync_copy(disbuf, dis_out.at[pl.ds(c * NP + base, ROWS_A)])


@functools.partial(
    pl.kernel,
    out_type=jax.ShapeDtypeStruct((2 * NP,), F32),
    mesh=_mesh,
    scratch_types=[
        pltpu.VMEM((128,), I32),
        pltpu.VMEM((128,), F32),
        pltpu.VMEM((128,), F32),
        pltpu.VMEM((128,), F32),
        pltpu.VMEM((ROWS_A,), F32),
        pltpu.VMEM((ROWS_A,), F32),
        pltpu.VMEM_SHARED((NP,), F32),
    ],
)
def k_deg(*refs):
    _k_deg(*refs)


def _k_mat_body(x_ref, wp_ref, out_ref):
    out_ref[...] = jnp.dot(x_ref[...], wp_ref[0], preferred_element_type=F32)


def k_mat(x, wp):
    return pl.pallas_call(
        _k_mat_body,
        grid=(2, 25),
        in_specs=[
            pl.BlockSpec((400, D), lambda c, r: (r, 0)),
            pl.BlockSpec((1, D, 128), lambda c, r: (c, 0, 0)),
        ],
        out_specs=pl.BlockSpec((400, 128), lambda c, r: (c * 25 + r, 0)),
        out_shape=jax.ShapeDtypeStruct((2 * N, 128), F32),
    )(x, wp)


def _k_mat2_body(x_ref, w_ref, out_ref):
    out_ref[...] = jnp.dot(x_ref[...], w_ref[...], preferred_element_type=F32)


def k_mat2(x, W):
    return pl.pallas_call(
        _k_mat2_body,
        grid=(25,),
        in_specs=[
            pl.BlockSpec((400, D), lambda r: (r, 0)),
            pl.BlockSpec((D, H), lambda r: (0, 0)),
        ],
        out_specs=pl.BlockSpec((400, 128), lambda r: (r, 0)),
        out_shape=jax.ShapeDtypeStruct((N, 128), F32),
    )(x, W)


def _k_msga(hcat, src2d, dst2d, ew2d, dis_all, zeros_a, acc_out_a,
            srcbuf, dstbuf, soffbuf, ewbuf, a1v, b1v, n1buf,
            rows_v, msg_a, sem, acc_as):
    c = lax.axis_index("c")
    s = lax.axis_index("s")

    pltpu.sync_copy(zeros_a, msg_a)

    def _zero(off, sz):
        pltpu.sync_copy(msg_a.at[pl.ds(0, sz)], acc_as.at[pl.ds(off, sz)])

    _rowshare(s, _zero)
    plsc.subcore_barrier()

    coff = c * N

    def body(i, _):
        blk = i * NSUB + s

        @pl.when(blk < NB)
        def _():
            pltpu.sync_copy(src2d.at[blk], srcbuf)
            pltpu.sync_copy(dst2d.at[blk], dstbuf)
            pltpu.sync_copy(ew2d.at[blk], ewbuf)
            for jj in range(8):
                sl = pl.ds(jj * 16, 16)
                soffbuf[sl] = srcbuf[sl] + coff
            cp0 = pltpu.async_copy(hcat.at[soffbuf], rows_v, sem)
            cp1 = pltpu.async_copy(dis_all.at[srcbuf], a1v, sem)
            cp2 = pltpu.async_copy(dis_all.at[dstbuf], b1v, sem)
            cp0.wait()
            cp1.wait()
            cp2.wait()
            for jj in range(8):
                sl = pl.ds(jj * 16, 16)
                n1buf[sl] = a1v[sl] * b1v[sl] * ewbuf[sl]

            def mbody(k, _):
                kc = (k // 16) * 16
                kl = jnp.full((16,), k % 16, I32)
                n1 = n1buf[pl.ds(kc, 16)][kl]
                for j in range(4):
                    slj = pl.ds(j * 16, 16)
                    slj1 = pl.ds(64 + j * 16, 16)
                    msg_a[k, slj] = rows_v[k, slj] * n1
                    msg_a[k, slj1] = rows_v[k, slj1] * n1
                return 0

            lax.fori_loop(0, 128, mbody, 0)
            pltpu.async_copy(msg_a, acc_as.at[dstbuf], sem, add=True).wait()
        return 0

    lax.fori_loop(0, (NB + NSUB - 1) // NSUB, body, 0)
    plsc.subcore_barrier()

    def _drain(off, sz):
        pltpu.sync_copy(acc_as.at[pl.ds(off, sz)], msg_a.at[pl.ds(0, sz)])
        pltpu.sync_copy(msg_a.at[pl.ds(0, sz)], acc_out_a.at[c, pl.ds(off, sz)])

    _rowshare(s, _drain)


@functools.partial(
    pl.kernel,
    out_type=jax.ShapeDtypeStruct((2, N, 128), F32),
    mesh=_mesh,
    scratch_types=[
        pltpu.VMEM((128,), I32),
        pltpu.VMEM((128,), I32),
        pltpu.VMEM((128,), I32),
        pltpu.VMEM((128,), F32),
        pltpu.VMEM((128,), F32),
        pltpu.VMEM((128,), F32),
        pltpu.VMEM((128,), F32),
        pltpu.VMEM((128, 128), F32),
        pltpu.VMEM((128, 128), F32),
        pltpu.SemaphoreType.DMA,
        pltpu.VMEM_SHARED((N, 128), F32),
    ],
)
def k_msga(*refs):
    _k_msga(*refs)


NBH = NB // 2


def _k_msgb(hfull, src2d, dst2d, ew2d, em2d, dis_all, zeros_b, acc_out_b,
            srcbuf, dstbuf, s2buf, d2buf, ewbuf, embuf,
            a2v, b2v, n2buf, rows_b, msg_b, sem, acc_bs):
    c = lax.axis_index("c")
    s = lax.axis_index("s")

    pltpu.sync_copy(zeros_b, msg_b)

    def _zero(off, sz):
        pltpu.sync_copy(msg_b.at[pl.ds(0, sz)], acc_bs.at[pl.ds(off, sz)])

    _rowshare(s, _zero)
    plsc.subcore_barrier()

    def body(i, _):
        k = i * NSUB + s

        @pl.when(k < NBH)
        def _():
            blk = c * NBH + k
            pltpu.sync_copy(src2d.at[blk], srcbuf)
            pltpu.sync_copy(dst2d.at[blk], dstbuf)
            pltpu.sync_copy(ew2d.at[blk], ewbuf)
            pltpu.sync_copy(em2d.at[blk], embuf)
            for jj in range(8):
                sl = pl.ds(jj * 16, 16)
                s2buf[sl] = srcbuf[sl] + NP
                d2buf[sl] = dstbuf[sl] + NP
            cp0 = pltpu.async_copy(hfull.at[srcbuf], rows_b, sem)
            cp1 = pltpu.async_copy(dis_all.at[s2buf], a2v, sem)
            cp2 = pltpu.async_copy(dis_all.at[d2buf], b2v, sem)
            cp0.wait()
            cp1.wait()
            cp2.wait()
            for jj in range(8):
                sl = pl.ds(jj * 16, 16)
                n2buf[sl] = a2v[sl] * b2v[sl] * (ewbuf[sl] * embuf[sl])

            def mbody(kk_, _):
                kc = (kk_ // 16) * 16
                kl = jnp.full((16,), kk_ % 16, I32)
                n2 = n2buf[pl.ds(kc, 16)][kl]
                for j in range(8):
                    slj = pl.ds(j * 16, 16)
                    msg_b[kk_, slj] = rows_b[kk_, slj] * n2
                return 0

            lax.fori_loop(0, 128, mbody, 0)
            pltpu.async_copy(msg_b, acc_bs.at[dstbuf], sem, add=True).wait()
        return 0

    lax.fori_loop(0, (NBH + NSUB - 1) // NSUB, body, 0)
    plsc.subcore_barrier()

    def _drain(off, sz):
        pltpu.sync_copy(acc_bs.at[pl.ds(off, sz)], msg_b.at[pl.ds(0, sz)])
        pltpu.sync_copy(msg_b.at[pl.ds(0, sz)], acc_out_b.at[c, pl.ds(off, sz)])

    _rowshare(s, _drain)


@functools.partial(
    pl.kernel,
    out_type=jax.ShapeDtypeStruct((2, N, 128), F32),
    mesh=_mesh,
    scratch_types=[
        pltpu.VMEM((128,), I32),
        pltpu.VMEM((128,), I32),
        pltpu.VMEM((128,), I32),
        pltpu.VMEM((128,), I32),
        pltpu.VMEM((128,), F32),
        pltpu.VMEM((128,), F32),
        pltpu.VMEM((128,), F32),
        pltpu.VMEM((128,), F32),
        pltpu.VMEM((128,), F32),
        pltpu.VMEM((128, 128), F32),
        pltpu.VMEM((128, 128), F32),
        pltpu.SemaphoreType.DMA,
        pltpu.VMEM_SHARED((N, 128), F32),
    ],
)
def k_msgb(*refs):
    _k_msgb(*refs)


def _k_out_body(acca_ref, accb_ref, n1_ref, n2_ref, z_ref, z1_ref, z2_ref):
    a0 = acca_ref[0]
    a1 = acca_ref[1]

    def halves(lo):
        return jnp.concatenate([a0[:, lo:lo + 64], a1[:, lo:lo + 64]], axis=1)

    z_ref[...] = jnp.maximum(halves(0), 0.0)
    z1_ref[...] = 0.9 * jnp.maximum(halves(64), 0.0) + 0.1 * n1_ref[...]
    zb = accb_ref[0] + accb_ref[1]
    z2_ref[...] = 0.9 * jnp.maximum(zb, 0.0) + 0.1 * n2_ref[...]


def k_out(acc_a, acc_b, noise1, noise2):
    o = jax.ShapeDtypeStruct((N, H), F32)
    return pl.pallas_call(
        _k_out_body,
        grid=(25,),
        in_specs=[
            pl.BlockSpec((2, 400, 128), lambda r: (0, r, 0)),
            pl.BlockSpec((2, 400, 128), lambda r: (0, r, 0)),
            pl.BlockSpec((400, 128), lambda r: (r, 0)),
            pl.BlockSpec((400, 128), lambda r: (r, 0)),
        ],
        out_specs=[
            pl.BlockSpec((400, 128), lambda r: (r, 0)),
            pl.BlockSpec((400, 128), lambda r: (r, 0)),
            pl.BlockSpec((400, 128), lambda r: (r, 0)),
        ],
        out_shape=(o, o, o),
    )(acc_a, acc_b, noise1, noise2)


def kernel(x, edge_index, edge_weight, W):
    kf = jax.random.key(42)
    ka, kb, kn1, kn2 = jax.random.split(kf, 4)
    feat_mask = (jax.random.uniform(ka, (1, D)) > 0.2).astype(x.dtype)
    edge_mask = (jax.random.uniform(kb, (E,)) > 0.2).astype(edge_weight.dtype)
    noise1 = jax.random.normal(kn1, (N, H), dtype=x.dtype)
    noise2 = jax.random.normal(kn2, (N, H), dtype=x.dtype)

    W1 = W * feat_mask[0][:, None]
    wp = jnp.stack([
        jnp.concatenate([W[:, 0:64], W1[:, 0:64]], axis=1),
        jnp.concatenate([W[:, 64:128], W1[:, 64:128]], axis=1),
    ])

    src2d = edge_index[0].astype(I32).reshape(NB, 128)
    dst2d = edge_index[1].astype(I32).reshape(NB, 128)
    ew2d = edge_weight.reshape(NB, 128)
    em2d = edge_mask.reshape(NB, 128)
    zeros1d = jnp.zeros((ROWS_A,), F32)
    zeros_a = jnp.zeros((128, 128), F32)

    dis_all = k_deg(dst2d, ew2d, em2d, zeros1d)
    hcat = k_mat(x, wp)
    hfull = k_mat2(x, W)
    acc_a = k_msga(hcat, src2d, dst2d, ew2d, dis_all, zeros_a)
    acc_b = k_msgb(hfull, src2d, dst2d, ew2d, em2d, dis_all, zeros_a)
    z, z1, z2 = k_out(acc_a, acc_b, noise1, noise2)
    return (z, z1, z2)


# speedup vs baseline: 9.5321x; 1.6182x over previous
import functools

import jax
import jax.numpy as jnp
from jax import lax
from jax.experimental import pallas as pl
from jax.experimental.pallas import tpu as pltpu
from jax.experimental.pallas import tpu_sc as plsc

N = 10000
E = 320000
D = 128
H = 128
NB = E // 128
NSUB = 16
NP = 10240
ROWS_A = NP // NSUB
RA = 632
F32 = jnp.float32
I32 = jnp.int32

_mesh = plsc.VectorSubcoreMesh(core_axis_name="c", subcore_axis_name="s",
                               num_cores=2, num_subcores=NSUB)


def _rsqrt16(x):
    b = lax.bitcast_convert_type(x, I32)
    y = lax.bitcast_convert_type(jnp.int32(0x5F3759DF) - (b >> 1), F32)
    for _ in range(3):
        y = y * (1.5 - 0.5 * x * y * y)
    return jnp.where(x > 0, y, 0.0)


def _rowshare(s, fn):
    @pl.when(s < 15)
    def _():
        for off, sz in ((0, 128), (128, 128), (256, 128), (384, 128),
                        (512, 120)):
            fn(s * RA + off, sz)

    @pl.when(s == 15)
    def _():
        for off, sz in ((0, 128), (128, 128), (256, 128), (384, 128),
                        (512, 8)):
            fn(15 * RA + off, sz)


def _k_deg(dst2d, ew2d, em2d, zeros1d, dis_out,
           dstbuf, ewbuf, embuf, wbuf, dbuf, disbuf, dacc):
    c = lax.axis_index("c")
    s = lax.axis_index("s")
    cf = c.astype(F32)

    pltpu.sync_copy(zeros1d, dbuf)
    pltpu.sync_copy(dbuf, dacc.at[pl.ds(s * ROWS_A, ROWS_A)])
    plsc.subcore_barrier()

    def body(i, _):
        blk = i * NSUB + s

        @pl.when(blk < NB)
        def _():
            pltpu.sync_copy(dst2d.at[blk], dstbuf)
            pltpu.sync_copy(ew2d.at[blk], ewbuf)
            pltpu.sync_copy(em2d.at[blk], embuf)
            for jj in range(8):
                sl = pl.ds(jj * 16, 16)
                e16 = ewbuf[sl]
                m16 = embuf[sl]
                wbuf[sl] = e16 * ((1.0 - cf) + cf * m16)
            pltpu.sync_copy(wbuf, dacc.at[dstbuf], add=True)
        return 0

    lax.fori_loop(0, (NB + NSUB - 1) // NSUB, body, 0)
    plsc.subcore_barrier()

    base = s * ROWS_A
    pltpu.sync_copy(dacc.at[pl.ds(base, ROWS_A)], dbuf)

    def rbody(j, _):
        sl = pl.ds(j * 16, 16)
        disbuf[sl] = _rsqrt16(dbuf[sl])
        return 0

    lax.fori_loop(0, ROWS_A // 16, rbody, 0)
    pltpu.sync_copy(disbuf, dis_out.at[pl.ds(c * NP + base, ROWS_A)])


@functools.partial(
    pl.kernel,
    out_type=jax.ShapeDtypeStruct((2 * NP,), F32),
    mesh=_mesh,
    scratch_types=[
        pltpu.VMEM((128,), I32),
        pltpu.VMEM((128,), F32),
        pltpu.VMEM((128,), F32),
        pltpu.VMEM((128,), F32),
        pltpu.VMEM((ROWS_A,), F32),
        pltpu.VMEM((ROWS_A,), F32),
        pltpu.VMEM_SHARED((NP,), F32),
    ],
)
def k_deg(*refs):
    _k_deg(*refs)


def _k_mat_body(x_ref, wp_ref, out_ref):
    out_ref[...] = jnp.dot(x_ref[...], wp_ref[0], preferred_element_type=F32)


def k_mat(x, wp):
    return pl.pallas_call(
        _k_mat_body,
        grid=(2, 25),
        in_specs=[
            pl.BlockSpec((400, D), lambda c, r: (r, 0)),
            pl.BlockSpec((1, D, 128), lambda c, r: (c, 0, 0)),
        ],
        out_specs=pl.BlockSpec((400, 128), lambda c, r: (c * 25 + r, 0)),
        out_shape=jax.ShapeDtypeStruct((2 * N, 128), F32),
    )(x, wp)


def _k_mat2_body(x_ref, w_ref, out_ref):
    out_ref[...] = jnp.dot(x_ref[...], w_ref[...], preferred_element_type=F32)


def k_mat2(x, W):
    return pl.pallas_call(
        _k_mat2_body,
        grid=(25,),
        in_specs=[
            pl.BlockSpec((400, D), lambda r: (r, 0)),
            pl.BlockSpec((D, H), lambda r: (0, 0)),
        ],
        out_specs=pl.BlockSpec((400, 128), lambda r: (r, 0)),
        out_shape=jax.ShapeDtypeStruct((N, 128), F32),
    )(x, W)


def _k_msga(hcat, src2d, dst2d, ew2d, dis_all, zeros_a, acc_out_a,
            srcbuf, dstbuf, soffbuf, ewbuf, a1v, b1v, n1buf,
            rows_v, msg_a, sem, acc_as):
    c = lax.axis_index("c")
    s = lax.axis_index("s")

    pltpu.sync_copy(zeros_a, msg_a)

    def _zero(off, sz):
        pltpu.sync_copy(msg_a.at[pl.ds(0, sz)], acc_as.at[pl.ds(off, sz)])

    _rowshare(s, _zero)
    plsc.subcore_barrier()

    coff = c * N

    def body(i, _):
        blk = i * NSUB + s

        @pl.when(blk < NB)
        def _():
            pltpu.sync_copy(src2d.at[blk], srcbuf)
            pltpu.sync_copy(dst2d.at[blk], dstbuf)
            pltpu.sync_copy(ew2d.at[blk], ewbuf)
            for jj in range(8):
                sl = pl.ds(jj * 16, 16)
                soffbuf[sl] = srcbuf[sl] + coff
            cp0 = pltpu.async_copy(hcat.at[soffbuf], rows_v, sem)
            cp1 = pltpu.async_copy(dis_all.at[srcbuf], a1v, sem)
            cp2 = pltpu.async_copy(dis_all.at[dstbuf], b1v, sem)
            cp0.wait()
            cp1.wait()
            cp2.wait()
            for jj in range(8):
                sl = pl.ds(jj * 16, 16)
                n1buf[sl] = a1v[sl] * b1v[sl] * ewbuf[sl]

            def mbody(g, _):
                g0 = g * 16
                n1c = n1buf[pl.ds(g0, 16)]
                for e in range(16):
                    k = g0 + e
                    n1 = n1c[jnp.full((16,), e, I32)]
                    for j in range(4):
                        slj = pl.ds(j * 16, 16)
                        slj1 = pl.ds(64 + j * 16, 16)
                        msg_a[k, slj] = rows_v[k, slj] * n1
                        msg_a[k, slj1] = rows_v[k, slj1] * n1
                return 0

            lax.fori_loop(0, 8, mbody, 0)
            pltpu.async_copy(msg_a, acc_as.at[dstbuf], sem, add=True).wait()
        return 0

    lax.fori_loop(0, (NB + NSUB - 1) // NSUB, body, 0)
    plsc.subcore_barrier()

    def _drain(off, sz):
        pltpu.sync_copy(acc_as.at[pl.ds(off, sz)], msg_a.at[pl.ds(0, sz)])
        pltpu.sync_copy(msg_a.at[pl.ds(0, sz)], acc_out_a.at[c, pl.ds(off, sz)])

    _rowshare(s, _drain)


@functools.partial(
    pl.kernel,
    out_type=jax.ShapeDtypeStruct((2, N, 128), F32),
    mesh=_mesh,
    scratch_types=[
        pltpu.VMEM((128,), I32),
        pltpu.VMEM((128,), I32),
        pltpu.VMEM((128,), I32),
        pltpu.VMEM((128,), F32),
        pltpu.VMEM((128,), F32),
        pltpu.VMEM((128,), F32),
        pltpu.VMEM((128,), F32),
        pltpu.VMEM((128, 128), F32),
        pltpu.VMEM((128, 128), F32),
        pltpu.SemaphoreType.DMA,
        pltpu.VMEM_SHARED((N, 128), F32),
    ],
)
def k_msga(*refs):
    _k_msga(*refs)


NBH = NB // 2


def _k_msgb(hfull, src2d, dst2d, ew2d, em2d, dis_all, zeros_b, acc_out_b,
            srcbuf, dstbuf, s2buf, d2buf, ewbuf, embuf,
            a2v, b2v, n2buf, rows_b, msg_b, sem, acc_bs):
    c = lax.axis_index("c")
    s = lax.axis_index("s")

    pltpu.sync_copy(zeros_b, msg_b)

    def _zero(off, sz):
        pltpu.sync_copy(msg_b.at[pl.ds(0, sz)], acc_bs.at[pl.ds(off, sz)])

    _rowshare(s, _zero)
    plsc.subcore_barrier()

    def body(i, _):
        k = i * NSUB + s

        @pl.when(k < NBH)
        def _():
            blk = c * NBH + k
            pltpu.sync_copy(src2d.at[blk], srcbuf)
            pltpu.sync_copy(dst2d.at[blk], dstbuf)
            pltpu.sync_copy(ew2d.at[blk], ewbuf)
            pltpu.sync_copy(em2d.at[blk], embuf)
            for jj in range(8):
                sl = pl.ds(jj * 16, 16)
                s2buf[sl] = srcbuf[sl] + NP
                d2buf[sl] = dstbuf[sl] + NP
            cp0 = pltpu.async_copy(hfull.at[srcbuf], rows_b, sem)
            cp1 = pltpu.async_copy(dis_all.at[s2buf], a2v, sem)
            cp2 = pltpu.async_copy(dis_all.at[d2buf], b2v, sem)
            cp0.wait()
            cp1.wait()
            cp2.wait()
            for jj in range(8):
                sl = pl.ds(jj * 16, 16)
                n2buf[sl] = a2v[sl] * b2v[sl] * (ewbuf[sl] * embuf[sl])

            def mbody(g, _):
                g0 = g * 16
                n2c = n2buf[pl.ds(g0, 16)]
                for e in range(16):
                    kk_ = g0 + e
                    n2 = n2c[jnp.full((16,), e, I32)]
                    for j in range(8):
                        slj = pl.ds(j * 16, 16)
                        msg_b[kk_, slj] = rows_b[kk_, slj] * n2
                return 0

            lax.fori_loop(0, 8, mbody, 0)
            pltpu.async_copy(msg_b, acc_bs.at[dstbuf], sem, add=True).wait()
        return 0

    lax.fori_loop(0, (NBH + NSUB - 1) // NSUB, body, 0)
    plsc.subcore_barrier()

    def _drain(off, sz):
        pltpu.sync_copy(acc_bs.at[pl.ds(off, sz)], msg_b.at[pl.ds(0, sz)])
        pltpu.sync_copy(msg_b.at[pl.ds(0, sz)], acc_out_b.at[c, pl.ds(off, sz)])

    _rowshare(s, _drain)


@functools.partial(
    pl.kernel,
    out_type=jax.ShapeDtypeStruct((2, N, 128), F32),
    mesh=_mesh,
    scratch_types=[
        pltpu.VMEM((128,), I32),
        pltpu.VMEM((128,), I32),
        pltpu.VMEM((128,), I32),
        pltpu.VMEM((128,), I32),
        pltpu.VMEM((128,), F32),
        pltpu.VMEM((128,), F32),
        pltpu.VMEM((128,), F32),
        pltpu.VMEM((128,), F32),
        pltpu.VMEM((128,), F32),
        pltpu.VMEM((128, 128), F32),
        pltpu.VMEM((128, 128), F32),
        pltpu.SemaphoreType.DMA,
        pltpu.VMEM_SHARED((N, 128), F32),
    ],
)
def k_msgb(*refs):
    _k_msgb(*refs)


def _k_out_body(acca_ref, accb_ref, n1_ref, n2_ref, z_ref, z1_ref, z2_ref):
    a0 = acca_ref[0]
    a1 = acca_ref[1]

    def halves(lo):
        return jnp.concatenate([a0[:, lo:lo + 64], a1[:, lo:lo + 64]], axis=1)

    z_ref[...] = jnp.maximum(halves(0), 0.0)
    z1_ref[...] = 0.9 * jnp.maximum(halves(64), 0.0) + 0.1 * n1_ref[...]
    zb = accb_ref[0] + accb_ref[1]
    z2_ref[...] = 0.9 * jnp.maximum(zb, 0.0) + 0.1 * n2_ref[...]


def k_out(acc_a, acc_b, noise1, noise2):
    o = jax.ShapeDtypeStruct((N, H), F32)
    return pl.pallas_call(
        _k_out_body,
        grid=(25,),
        in_specs=[
            pl.BlockSpec((2, 400, 128), lambda r: (0, r, 0)),
            pl.BlockSpec((2, 400, 128), lambda r: (0, r, 0)),
            pl.BlockSpec((400, 128), lambda r: (r, 0)),
            pl.BlockSpec((400, 128), lambda r: (r, 0)),
        ],
        out_specs=[
            pl.BlockSpec((400, 128), lambda r: (r, 0)),
            pl.BlockSpec((400, 128), lambda r: (r, 0)),
            pl.BlockSpec((400, 128), lambda r: (r, 0)),
        ],
        out_shape=(o, o, o),
    )(acc_a, acc_b, noise1, noise2)


def kernel(x, edge_index, edge_weight, W):
    kf = jax.random.key(42)
    ka, kb, kn1, kn2 = jax.random.split(kf, 4)
    feat_mask = (jax.random.uniform(ka, (1, D)) > 0.2).astype(x.dtype)
    edge_mask = (jax.random.uniform(kb, (E,)) > 0.2).astype(edge_weight.dtype)
    noise1 = jax.random.normal(kn1, (N, H), dtype=x.dtype)
    noise2 = jax.random.normal(kn2, (N, H), dtype=x.dtype)

    W1 = W * feat_mask[0][:, None]
    wp = jnp.stack([
        jnp.concatenate([W[:, 0:64], W1[:, 0:64]], axis=1),
        jnp.concatenate([W[:, 64:128], W1[:, 64:128]], axis=1),
    ])

    src2d = edge_index[0].astype(I32).reshape(NB, 128)
    dst2d = edge_index[1].astype(I32).reshape(NB, 128)
    ew2d = edge_weight.reshape(NB, 128)
    em2d = edge_mask.reshape(NB, 128)
    zeros1d = jnp.zeros((ROWS_A,), F32)
    zeros_a = jnp.zeros((128, 128), F32)

    dis_all = k_deg(dst2d, ew2d, em2d, zeros1d)
    hcat = k_mat(x, wp)
    hfull = k_mat2(x, W)
    acc_a = k_msga(hcat, src2d, dst2d, ew2d, dis_all, zeros_a)
    acc_b = k_msgb(hfull, src2d, dst2d, ew2d, em2d, dis_all, zeros_a)
    z, z1, z2 = k_out(acc_a, acc_b, noise1, noise2)
    return (z, z1, z2)
